# Initial kernel scaffold; baseline (speedup 1.0000x reference)
import jax
import jax.numpy as jnp
from jax.experimental import pallas as pl


def kernel(hidden_states, position_ids, cos, sin, visual_token_start, visual_token_end):
    raise NotImplementedError("write your pallas kernel here")


# speedup vs baseline: 1.0716x; 1.0716x over previous
import jax
import jax.numpy as jnp
from jax.experimental import pallas as pl

_T, _N, _D, _HD = 32, 256, 2048, 128
_VS, _VE, _S = 64, 8256, 8320
_K = 5
_C = 64
_NSEG = 8
_TAU = 0.8
_EPS = 1e-6

_HI = jax.lax.Precision.HIGHEST


def _gram(f):
    return jax.lax.dot_general(
        f, f, (((1,), (1,)), ((), ())),
        precision=jax.lax.Precision.DEFAULT,
        preferred_element_type=jnp.float32)


def _dpc_labels(f, k, c):
    p = f.shape[0]
    sq = jnp.sum(f * f, axis=1)
    g = _gram(f)
    d2 = jnp.maximum(sq[:, None] + sq[None, :] - 2.0 * g, 0.0)
    cols = jax.lax.broadcasted_iota(jnp.int32, (p, p), 1)
    work = d2
    acc = jnp.zeros((p,), jnp.float32)
    for _ in range(k):
        m = jnp.min(work, axis=1)
        acc = acc + m
        first = jnp.min(jnp.where(work == m[:, None], cols, p), axis=1)
        work = jnp.where(cols == first[:, None], jnp.inf, work)
    density = jnp.exp(-(acc / float(k)))
    big = jnp.max(d2) + 1.0
    higher = density[None, :] > density[:, None]
    delta = jnp.min(jnp.where(higher, d2, big), axis=1)
    delta = jnp.where(density >= jnp.max(density), jnp.max(d2, axis=1), delta)
    score = density * delta
    iot = jax.lax.broadcasted_iota(jnp.int32, (1, p), 1)

    def _step(s, carry):
        wsc, slot = carry
        m = jnp.max(wsc)
        idx = jnp.min(jnp.where(wsc == m, iot, p))
        slot = jnp.where(iot == idx, s, slot)
        wsc = jnp.where(iot == idx, -1.0, wsc)
        return wsc, slot

    _, slot = jax.lax.fori_loop(
        0, c, _step, (score[None, :], jnp.full((1, p), p, jnp.int32)))
    is_c = slot < c
    cand = jnp.where(is_c, d2, jnp.inf)
    minv = jnp.min(cand, axis=1)
    lbl = jnp.min(jnp.where(is_c & (cand == minv[:, None]),
                            jnp.broadcast_to(slot, (p, p)), p), axis=1)
    return lbl


def _stats_kernel(xa_ref, xb_ref, ms_ref, ff_ref, sim_ref):
    t = pl.program_id(0)
    xa = xa_ref[0]
    xb = xb_ref[0]

    @pl.when(t == 0)
    def _():
        ms_ref[...] = xa

    @pl.when(t > 0)
    def _():
        ms_ref[...] = ms_ref[...] + xa

    ff_ref[0, 0] = jnp.sum(xa, axis=0) / float(_N)
    na = jnp.sqrt(jnp.sum(xa * xa, axis=1))
    nb = jnp.sqrt(jnp.sum(xb * xb, axis=1))
    xan = xa / (na[:, None] + _EPS)
    xbn = xb / (nb[:, None] + _EPS)
    sim_ref[0, 0] = jnp.sum(xan * xbn, axis=1)


def _wstatic_kernel(ff_ref, sim_ref, w_ref):
    lbl = _dpc_labels(ff_ref[:, 0, :], _K, _NSEG)
    same = lbl[0:_T - 1] == lbl[1:_T]
    sim = sim_ref[0:_T - 1, 0, :]
    sim_eff = jnp.where(same[:, None], sim, 1.0)
    static_score = jnp.min(sim_eff, axis=0)
    w_ref[0, 0] = (static_score > _TAU).astype(jnp.float32)


def _merge_kernel(x_ref, ms_ref, w_ref, pos_ref, cos_ref, sin_ref,
                  h_ref, p_ref, c_ref, s_ref):
    gidx = pl.program_id(0)
    is_static = gidx == _T
    f = jnp.where(is_static, ms_ref[...] / float(_T), x_ref[0])
    wv = w_ref[0, 0]
    w = jnp.where(is_static, wv, 1.0 - wv)
    lbl = _dpc_labels(f, _K, _C)
    seg = jax.lax.broadcasted_iota(jnp.int32, (_C, _N), 0)
    onehot = (lbl[None, :] == seg).astype(jnp.float32)
    den = jnp.sum(onehot * w[None, :], axis=1) + _EPS
    for c0 in range(0, _D, 512):
        fwc = f[:, c0:c0 + 512] * w[:, None]
        shc = jax.lax.dot_general(onehot, fwc, (((1,), (0,)), ((), ())),
                                  precision=_HI,
                                  preferred_element_type=jnp.float32)
        h_ref[0, :, c0:c0 + 512] = shc / den[:, None]
    posw = pos_ref[0] * w[None, :]
    sp = jax.lax.dot_general(posw, onehot, (((1,), (1,)), ((), ())),
                             precision=_HI, preferred_element_type=jnp.float32)
    p_ref[0] = sp / den[None, :]
    for i in range(3):
        cw = cos_ref[i, 0] * w[:, None]
        sc = jax.lax.dot_general(onehot, cw, (((1,), (0,)), ((), ())),
                                 precision=_HI,
                                 preferred_element_type=jnp.float32)
        c_ref[0, i] = sc / den[:, None]
        sw = sin_ref[i, 0] * w[:, None]
        ss = jax.lax.dot_general(onehot, sw, (((1,), (0,)), ((), ())),
                                 precision=_HI,
                                 preferred_element_type=jnp.float32)
        s_ref[0, i] = ss / den[:, None]


def kernel(hidden_states, position_ids, cos, sin,
           visual_token_start, visual_token_end):
    vs = visual_token_start
    ve = visual_token_end
    nv = _VE - _VS
    nt = _S - _VE
    f32 = jnp.float32
    x3 = jax.lax.dynamic_slice(
        hidden_states, (0, vs, 0), (1, nv, _D))[0].reshape(_T, _N, _D)
    posv = (jax.lax.dynamic_slice(position_ids, (0, 0, vs), (3, 1, nv))[:, 0]
            .astype(f32).reshape(3, _T, _N).transpose(1, 0, 2))
    cosv = jax.lax.dynamic_slice(
        cos, (0, 0, vs, 0), (3, 1, nv, _HD))[:, 0].reshape(3, _T, _N, _HD)
    sinv = jax.lax.dynamic_slice(
        sin, (0, 0, vs, 0), (3, 1, nv, _HD))[:, 0].reshape(3, _T, _N, _HD)

    ms_sum, ff, sims = pl.pallas_call(
        _stats_kernel,
        grid=(_T,),
        in_specs=[
            pl.BlockSpec((1, _N, _D), lambda t: (t, 0, 0)),
            pl.BlockSpec((1, _N, _D), lambda t: (jnp.minimum(t + 1, _T - 1), 0, 0)),
        ],
        out_specs=[
            pl.BlockSpec((_N, _D), lambda t: (0, 0)),
            pl.BlockSpec((1, 1, _D), lambda t: (t, 0, 0)),
            pl.BlockSpec((1, 1, _N), lambda t: (t, 0, 0)),
        ],
        out_shape=[
            jax.ShapeDtypeStruct((_N, _D), f32),
            jax.ShapeDtypeStruct((_T, 1, _D), f32),
            jax.ShapeDtypeStruct((_T, 1, _N), f32),
        ],
    )(x3, x3)

    w_static = pl.pallas_call(
        _wstatic_kernel,
        out_shape=jax.ShapeDtypeStruct((1, 1, _N), f32),
    )(ff, sims)

    h_o, p_o, c_o, s_o = pl.pallas_call(
        _merge_kernel,
        grid=(_T + 1,),
        in_specs=[
            pl.BlockSpec((1, _N, _D), lambda g: (jnp.minimum(g, _T - 1), 0, 0)),
            pl.BlockSpec((_N, _D), lambda g: (0, 0)),
            pl.BlockSpec((1, 1, _N), lambda g: (0, 0, 0)),
            pl.BlockSpec((1, 3, _N), lambda g: (g % _T, 0, 0)),
            pl.BlockSpec((3, 1, _N, _HD), lambda g: (0, g % _T, 0, 0)),
            pl.BlockSpec((3, 1, _N, _HD), lambda g: (0, g % _T, 0, 0)),
        ],
        out_specs=[
            pl.BlockSpec((1, _C, _D), lambda g: (g, 0, 0)),
            pl.BlockSpec((1, 3, _C), lambda g: (g, 0, 0)),
            pl.BlockSpec((1, 3, _C, _HD), lambda g: (g, 0, 0, 0)),
            pl.BlockSpec((1, 3, _C, _HD), lambda g: (g, 0, 0, 0)),
        ],
        out_shape=[
            jax.ShapeDtypeStruct((_T + 1, _C, _D), f32),
            jax.ShapeDtypeStruct((_T + 1, 3, _C), f32),
            jax.ShapeDtypeStruct((_T + 1, 3, _C, _HD), f32),
            jax.ShapeDtypeStruct((_T + 1, 3, _C, _HD), f32),
        ],
    )(x3, ms_sum, w_static, posv, cosv, sinv)

    head_h = hidden_states[0, :_VS, :]
    tail_h = jax.lax.dynamic_slice(hidden_states, (0, ve, 0), (1, nt, _D))[0]
    head_p = position_ids[:, 0, :_VS].astype(f32)
    tail_p = jax.lax.dynamic_slice(
        position_ids, (0, 0, ve), (3, 1, nt))[:, 0].astype(f32)
    head_c = cos[:, 0, :_VS, :]
    tail_c = jax.lax.dynamic_slice(cos, (0, 0, ve, 0), (3, 1, nt, _HD))[:, 0]
    head_s = sin[:, 0, :_VS, :]
    tail_s = jax.lax.dynamic_slice(sin, (0, 0, ve, 0), (3, 1, nt, _HD))[:, 0]

    merged_hidden = jnp.concatenate(
        [head_h, h_o[_T], h_o[:_T].reshape(_T * _C, _D), tail_h], axis=0)
    merged_pos = jnp.concatenate(
        [head_p, p_o[_T], p_o[:_T].transpose(1, 0, 2).reshape(3, _T * _C),
         tail_p], axis=1)
    merged_cos = jnp.concatenate(
        [head_c, c_o[_T],
         c_o[:_T].transpose(1, 0, 2, 3).reshape(3, _T * _C, _HD), tail_c],
        axis=1)
    merged_sin = jnp.concatenate(
        [head_s, s_o[_T],
         s_o[:_T].transpose(1, 0, 2, 3).reshape(3, _T * _C, _HD), tail_s],
        axis=1)
    return merged_hidden, merged_pos, merged_cos, merged_sin


# speedup vs baseline: 1.7407x; 1.6243x over previous
import jax
import jax.numpy as jnp
from jax.experimental import pallas as pl

_T, _N, _D, _HD = 32, 256, 2048, 128
_VS, _VE, _S = 64, 8256, 8320
_K = 5
_C = 64
_NSEG = 8
_TAU = 0.8
_EPS = 1e-6

_HI = jax.lax.Precision.HIGHEST


def _gram(f):
    return jax.lax.dot_general(
        f, f, (((1,), (1,)), ((), ())),
        precision=jax.lax.Precision.DEFAULT,
        preferred_element_type=jnp.float32)


def _dpc_labels(f, k, c):
    p = f.shape[0]
    sq = jnp.sum(f * f, axis=1)
    g = _gram(f)
    d2 = jnp.maximum(sq[:, None] + sq[None, :] - 2.0 * g, 0.0)
    cols = jax.lax.broadcasted_iota(jnp.int32, (p, p), 1)
    work = d2
    acc = jnp.zeros((p,), jnp.float32)
    for _ in range(k):
        m = jnp.min(work, axis=1)
        acc = acc + m
        first = jnp.min(jnp.where(work == m[:, None], cols, p), axis=1)
        work = jnp.where(cols == first[:, None], jnp.inf, work)
    density = jnp.exp(-(acc / float(k)))
    big = jnp.max(d2) + 1.0
    higher = density[None, :] > density[:, None]
    delta = jnp.min(jnp.where(higher, d2, big), axis=1)
    delta = jnp.where(density >= jnp.max(density), jnp.max(d2, axis=1), delta)
    score = density * delta
    rows = jax.lax.broadcasted_iota(jnp.int32, (p, p), 0)
    sk = score[:, None]
    sj = score[None, :]
    beats = (sk > sj) | ((sk == sj) & (rows < cols))
    slot = jnp.sum(beats.astype(jnp.float32), axis=0,
                   keepdims=True).astype(jnp.int32)
    is_c = slot < c
    cand = jnp.where(is_c, d2, jnp.inf)
    minv = jnp.min(cand, axis=1)
    lbl = jnp.min(jnp.where(is_c & (cand == minv[:, None]),
                            jnp.broadcast_to(slot, (p, p)), p), axis=1)
    return lbl


def _stats_kernel(xa_ref, xb_ref, ms_ref, ff_ref, sim_ref):
    t = pl.program_id(0)
    xa = xa_ref[0]
    xb = xb_ref[0]

    @pl.when(t == 0)
    def _():
        ms_ref[...] = xa

    @pl.when(t > 0)
    def _():
        ms_ref[...] = ms_ref[...] + xa

    ff_ref[0, 0] = jnp.sum(xa, axis=0) / float(_N)
    na = jnp.sqrt(jnp.sum(xa * xa, axis=1))
    nb = jnp.sqrt(jnp.sum(xb * xb, axis=1))
    xan = xa / (na[:, None] + _EPS)
    xbn = xb / (nb[:, None] + _EPS)
    sim_ref[0, 0] = jnp.sum(xan * xbn, axis=1)


def _wstatic_kernel(ff_ref, sim_ref, w_ref):
    lbl = _dpc_labels(ff_ref[:, 0, :], _K, _NSEG)
    same = lbl[0:_T - 1] == lbl[1:_T]
    sim = sim_ref[0:_T - 1, 0, :]
    sim_eff = jnp.where(same[:, None], sim, 1.0)
    static_score = jnp.min(sim_eff, axis=0)
    w_ref[0, 0] = (static_score > _TAU).astype(jnp.float32)


def _merge_kernel(x_ref, ms_ref, w_ref, pos_ref, cos_ref, sin_ref,
                  h_ref, p_ref, c_ref, s_ref):
    gidx = pl.program_id(0)
    is_static = gidx == _T
    f = jnp.where(is_static, ms_ref[...] / float(_T), x_ref[0])
    wv = w_ref[0, 0]
    w = jnp.where(is_static, wv, 1.0 - wv)
    lbl = _dpc_labels(f, _K, _C)
    seg = jax.lax.broadcasted_iota(jnp.int32, (_C, _N), 0)
    onehot = (lbl[None, :] == seg).astype(jnp.float32)
    den = jnp.sum(onehot * w[None, :], axis=1) + _EPS
    for c0 in range(0, _D, 512):
        fwc = f[:, c0:c0 + 512] * w[:, None]
        shc = jax.lax.dot_general(onehot, fwc, (((1,), (0,)), ((), ())),
                                  precision=_HI,
                                  preferred_element_type=jnp.float32)
        h_ref[0, :, c0:c0 + 512] = shc / den[:, None]
    posw = pos_ref[0] * w[None, :]
    sp = jax.lax.dot_general(posw, onehot, (((1,), (1,)), ((), ())),
                             precision=_HI, preferred_element_type=jnp.float32)
    p_ref[0] = sp / den[None, :]
    for i in range(3):
        cw = cos_ref[i, 0] * w[:, None]
        sc = jax.lax.dot_general(onehot, cw, (((1,), (0,)), ((), ())),
                                 precision=_HI,
                                 preferred_element_type=jnp.float32)
        c_ref[0, i] = sc / den[:, None]
        sw = sin_ref[i, 0] * w[:, None]
        ss = jax.lax.dot_general(onehot, sw, (((1,), (0,)), ((), ())),
                                 precision=_HI,
                                 preferred_element_type=jnp.float32)
        s_ref[0, i] = ss / den[:, None]


def kernel(hidden_states, position_ids, cos, sin,
           visual_token_start, visual_token_end):
    vs = visual_token_start
    ve = visual_token_end
    nv = _VE - _VS
    nt = _S - _VE
    f32 = jnp.float32
    x3 = jax.lax.dynamic_slice(
        hidden_states, (0, vs, 0), (1, nv, _D))[0].reshape(_T, _N, _D)
    posv = (jax.lax.dynamic_slice(position_ids, (0, 0, vs), (3, 1, nv))[:, 0]
            .astype(f32).reshape(3, _T, _N).transpose(1, 0, 2))
    cosv = jax.lax.dynamic_slice(
        cos, (0, 0, vs, 0), (3, 1, nv, _HD))[:, 0].reshape(3, _T, _N, _HD)
    sinv = jax.lax.dynamic_slice(
        sin, (0, 0, vs, 0), (3, 1, nv, _HD))[:, 0].reshape(3, _T, _N, _HD)

    ms_sum, ff, sims = pl.pallas_call(
        _stats_kernel,
        grid=(_T,),
        in_specs=[
            pl.BlockSpec((1, _N, _D), lambda t: (t, 0, 0)),
            pl.BlockSpec((1, _N, _D), lambda t: (jnp.minimum(t + 1, _T - 1), 0, 0)),
        ],
        out_specs=[
            pl.BlockSpec((_N, _D), lambda t: (0, 0)),
            pl.BlockSpec((1, 1, _D), lambda t: (t, 0, 0)),
            pl.BlockSpec((1, 1, _N), lambda t: (t, 0, 0)),
        ],
        out_shape=[
            jax.ShapeDtypeStruct((_N, _D), f32),
            jax.ShapeDtypeStruct((_T, 1, _D), f32),
            jax.ShapeDtypeStruct((_T, 1, _N), f32),
        ],
    )(x3, x3)

    w_static = pl.pallas_call(
        _wstatic_kernel,
        out_shape=jax.ShapeDtypeStruct((1, 1, _N), f32),
    )(ff, sims)

    h_o, p_o, c_o, s_o = pl.pallas_call(
        _merge_kernel,
        grid=(_T + 1,),
        in_specs=[
            pl.BlockSpec((1, _N, _D), lambda g: (jnp.minimum(g, _T - 1), 0, 0)),
            pl.BlockSpec((_N, _D), lambda g: (0, 0)),
            pl.BlockSpec((1, 1, _N), lambda g: (0, 0, 0)),
            pl.BlockSpec((1, 3, _N), lambda g: (g % _T, 0, 0)),
            pl.BlockSpec((3, 1, _N, _HD), lambda g: (0, g % _T, 0, 0)),
            pl.BlockSpec((3, 1, _N, _HD), lambda g: (0, g % _T, 0, 0)),
        ],
        out_specs=[
            pl.BlockSpec((1, _C, _D), lambda g: (g, 0, 0)),
            pl.BlockSpec((1, 3, _C), lambda g: (g, 0, 0)),
            pl.BlockSpec((1, 3, _C, _HD), lambda g: (g, 0, 0, 0)),
            pl.BlockSpec((1, 3, _C, _HD), lambda g: (g, 0, 0, 0)),
        ],
        out_shape=[
            jax.ShapeDtypeStruct((_T + 1, _C, _D), f32),
            jax.ShapeDtypeStruct((_T + 1, 3, _C), f32),
            jax.ShapeDtypeStruct((_T + 1, 3, _C, _HD), f32),
            jax.ShapeDtypeStruct((_T + 1, 3, _C, _HD), f32),
        ],
    )(x3, ms_sum, w_static, posv, cosv, sinv)

    head_h = hidden_states[0, :_VS, :]
    tail_h = jax.lax.dynamic_slice(hidden_states, (0, ve, 0), (1, nt, _D))[0]
    head_p = position_ids[:, 0, :_VS].astype(f32)
    tail_p = jax.lax.dynamic_slice(
        position_ids, (0, 0, ve), (3, 1, nt))[:, 0].astype(f32)
    head_c = cos[:, 0, :_VS, :]
    tail_c = jax.lax.dynamic_slice(cos, (0, 0, ve, 0), (3, 1, nt, _HD))[:, 0]
    head_s = sin[:, 0, :_VS, :]
    tail_s = jax.lax.dynamic_slice(sin, (0, 0, ve, 0), (3, 1, nt, _HD))[:, 0]

    merged_hidden = jnp.concatenate(
        [head_h, h_o[_T], h_o[:_T].reshape(_T * _C, _D), tail_h], axis=0)
    merged_pos = jnp.concatenate(
        [head_p, p_o[_T], p_o[:_T].transpose(1, 0, 2).reshape(3, _T * _C),
         tail_p], axis=1)
    merged_cos = jnp.concatenate(
        [head_c, c_o[_T],
         c_o[:_T].transpose(1, 0, 2, 3).reshape(3, _T * _C, _HD), tail_c],
        axis=1)
    merged_sin = jnp.concatenate(
        [head_s, s_o[_T],
         s_o[:_T].transpose(1, 0, 2, 3).reshape(3, _T * _C, _HD), tail_s],
        axis=1)
    return merged_hidden, merged_pos, merged_cos, merged_sin


# speedup vs baseline: 6.7939x; 3.9030x over previous
import jax
import jax.numpy as jnp
from jax.experimental import pallas as pl

_T, _N, _D, _HD = 32, 256, 2048, 128
_VS, _VE, _S = 64, 8256, 8320
_K = 5
_C = 64
_NSEG = 8
_TAU = 0.8
_EPS = 1e-6

_HI = jax.lax.Precision.HIGHEST


def _gram(f):
    return jax.lax.dot_general(
        f, f, (((1,), (1,)), ((), ())),
        precision=jax.lax.Precision.DEFAULT,
        preferred_element_type=jnp.float32)


def _dpc_labels(f, k, c):
    p = f.shape[0]
    sq = jnp.sum(f * f, axis=1)
    g = _gram(f)
    d2 = jnp.maximum(sq[:, None] + sq[None, :] - 2.0 * g, 0.0)
    rows = jax.lax.broadcasted_iota(jnp.int32, (p, p), 0)
    cols = jax.lax.broadcasted_iota(jnp.int32, (p, p), 1)
    work = d2
    acc = jnp.zeros((1, p), jnp.float32)
    for _ in range(k):
        m = jnp.min(work, axis=0, keepdims=True)
        acc = acc + m
        first = jnp.min(jnp.where(work == m, rows, p), axis=0, keepdims=True)
        work = jnp.where(rows == first, jnp.inf, work)
    density = jnp.exp(-(acc / float(k)))
    big = jnp.max(d2) + 1.0
    density_col = density.reshape(p, 1)
    higher = density_col > density
    delta = jnp.min(jnp.where(higher, d2, big), axis=0)
    dmax = jnp.max(d2, axis=0)
    delta = jnp.where(density[0] >= jnp.max(density), dmax, delta)
    score = (density[0] * delta)[None, :]
    score_col = score.reshape(p, 1)
    beats = (score_col > score) | ((score_col == score) & (rows < cols))
    slot = jnp.sum(beats.astype(jnp.float32), axis=0,
                   keepdims=True).astype(jnp.int32)
    slot_col = slot.reshape(p, 1)
    is_c_col = slot_col < c
    cand = jnp.where(is_c_col, d2, jnp.inf)
    minv = jnp.min(cand, axis=0, keepdims=True)
    lbl = jnp.min(jnp.where(is_c_col & (cand == minv),
                            jnp.broadcast_to(slot_col, (p, p)), p), axis=0)
    return lbl


def _stats_kernel(xa_ref, xb_ref, ms_ref, ff_ref, sim_ref):
    t = pl.program_id(0)
    xa = xa_ref[0]
    xb = xb_ref[0]

    @pl.when(t == 0)
    def _():
        ms_ref[...] = xa

    @pl.when(t > 0)
    def _():
        ms_ref[...] = ms_ref[...] + xa

    ff_ref[0, 0] = jnp.sum(xa, axis=0) / float(_N)
    na = jnp.sqrt(jnp.sum(xa * xa, axis=1))
    nb = jnp.sqrt(jnp.sum(xb * xb, axis=1))
    xan = xa / (na[:, None] + _EPS)
    xbn = xb / (nb[:, None] + _EPS)
    sim_ref[0, 0] = jnp.sum(xan * xbn, axis=1)


def _wstatic_kernel(ff_ref, sim_ref, w_ref):
    lbl = _dpc_labels(ff_ref[:, 0, :], _K, _NSEG)
    lbl_col = lbl[None, :].reshape(_T, 1)
    same_col = lbl_col[0:_T - 1] == lbl_col[1:_T]
    sim = sim_ref[0:_T - 1, 0, :]
    sim_eff = jnp.where(same_col, sim, 1.0)
    static_score = jnp.min(sim_eff, axis=0)
    w_ref[0, 0] = (static_score > _TAU).astype(jnp.float32)


def _merge_kernel(x_ref, ms_ref, w_ref, pos_ref, cos_ref, sin_ref,
                  h_ref, p_ref, c_ref, s_ref):
    gidx = pl.program_id(0)
    is_static = gidx == _T
    f = jnp.where(is_static, ms_ref[...] / float(_T), x_ref[0])
    wv = w_ref[0, 0]
    w = jnp.where(is_static, wv, 1.0 - wv)
    lbl = _dpc_labels(f, _K, _C)
    seg = jax.lax.broadcasted_iota(jnp.int32, (_C, _N), 0)
    onehot = (lbl[None, :] == seg).astype(jnp.float32)
    den = jnp.sum(onehot * w[None, :], axis=1) + _EPS
    for c0 in range(0, _D, 512):
        fwc = f[:, c0:c0 + 512] * w[:, None]
        shc = jax.lax.dot_general(onehot, fwc, (((1,), (0,)), ((), ())),
                                  precision=_HI,
                                  preferred_element_type=jnp.float32)
        h_ref[0, :, c0:c0 + 512] = shc / den[:, None]
    posw = pos_ref[0] * w[None, :]
    sp = jax.lax.dot_general(posw, onehot, (((1,), (1,)), ((), ())),
                             precision=_HI, preferred_element_type=jnp.float32)
    p_ref[0] = sp / den[None, :]
    for i in range(3):
        cw = cos_ref[i, 0] * w[:, None]
        sc = jax.lax.dot_general(onehot, cw, (((1,), (0,)), ((), ())),
                                 precision=_HI,
                                 preferred_element_type=jnp.float32)
        c_ref[0, i] = sc / den[:, None]
        sw = sin_ref[i, 0] * w[:, None]
        ss = jax.lax.dot_general(onehot, sw, (((1,), (0,)), ((), ())),
                                 precision=_HI,
                                 preferred_element_type=jnp.float32)
        s_ref[0, i] = ss / den[:, None]


def kernel(hidden_states, position_ids, cos, sin,
           visual_token_start, visual_token_end):
    vs = visual_token_start
    ve = visual_token_end
    nv = _VE - _VS
    nt = _S - _VE
    f32 = jnp.float32
    x3 = jax.lax.dynamic_slice(
        hidden_states, (0, vs, 0), (1, nv, _D))[0].reshape(_T, _N, _D)
    posv = (jax.lax.dynamic_slice(position_ids, (0, 0, vs), (3, 1, nv))[:, 0]
            .astype(f32).reshape(3, _T, _N).transpose(1, 0, 2))
    cosv = jax.lax.dynamic_slice(
        cos, (0, 0, vs, 0), (3, 1, nv, _HD))[:, 0].reshape(3, _T, _N, _HD)
    sinv = jax.lax.dynamic_slice(
        sin, (0, 0, vs, 0), (3, 1, nv, _HD))[:, 0].reshape(3, _T, _N, _HD)

    ms_sum, ff, sims = pl.pallas_call(
        _stats_kernel,
        grid=(_T,),
        in_specs=[
            pl.BlockSpec((1, _N, _D), lambda t: (t, 0, 0)),
            pl.BlockSpec((1, _N, _D), lambda t: (jnp.minimum(t + 1, _T - 1), 0, 0)),
        ],
        out_specs=[
            pl.BlockSpec((_N, _D), lambda t: (0, 0)),
            pl.BlockSpec((1, 1, _D), lambda t: (t, 0, 0)),
            pl.BlockSpec((1, 1, _N), lambda t: (t, 0, 0)),
        ],
        out_shape=[
            jax.ShapeDtypeStruct((_N, _D), f32),
            jax.ShapeDtypeStruct((_T, 1, _D), f32),
            jax.ShapeDtypeStruct((_T, 1, _N), f32),
        ],
    )(x3, x3)

    w_static = pl.pallas_call(
        _wstatic_kernel,
        out_shape=jax.ShapeDtypeStruct((1, 1, _N), f32),
    )(ff, sims)

    h_o, p_o, c_o, s_o = pl.pallas_call(
        _merge_kernel,
        grid=(_T + 1,),
        in_specs=[
            pl.BlockSpec((1, _N, _D), lambda g: (jnp.minimum(g, _T - 1), 0, 0)),
            pl.BlockSpec((_N, _D), lambda g: (0, 0)),
            pl.BlockSpec((1, 1, _N), lambda g: (0, 0, 0)),
            pl.BlockSpec((1, 3, _N), lambda g: (g % _T, 0, 0)),
            pl.BlockSpec((3, 1, _N, _HD), lambda g: (0, g % _T, 0, 0)),
            pl.BlockSpec((3, 1, _N, _HD), lambda g: (0, g % _T, 0, 0)),
        ],
        out_specs=[
            pl.BlockSpec((1, _C, _D), lambda g: (g, 0, 0)),
            pl.BlockSpec((1, 3, _C), lambda g: (g, 0, 0)),
            pl.BlockSpec((1, 3, _C, _HD), lambda g: (g, 0, 0, 0)),
            pl.BlockSpec((1, 3, _C, _HD), lambda g: (g, 0, 0, 0)),
        ],
        out_shape=[
            jax.ShapeDtypeStruct((_T + 1, _C, _D), f32),
            jax.ShapeDtypeStruct((_T + 1, 3, _C), f32),
            jax.ShapeDtypeStruct((_T + 1, 3, _C, _HD), f32),
            jax.ShapeDtypeStruct((_T + 1, 3, _C, _HD), f32),
        ],
    )(x3, ms_sum, w_static, posv, cosv, sinv)

    head_h = hidden_states[0, :_VS, :]
    tail_h = jax.lax.dynamic_slice(hidden_states, (0, ve, 0), (1, nt, _D))[0]
    head_p = position_ids[:, 0, :_VS].astype(f32)
    tail_p = jax.lax.dynamic_slice(
        position_ids, (0, 0, ve), (3, 1, nt))[:, 0].astype(f32)
    head_c = cos[:, 0, :_VS, :]
    tail_c = jax.lax.dynamic_slice(cos, (0, 0, ve, 0), (3, 1, nt, _HD))[:, 0]
    head_s = sin[:, 0, :_VS, :]
    tail_s = jax.lax.dynamic_slice(sin, (0, 0, ve, 0), (3, 1, nt, _HD))[:, 0]

    merged_hidden = jnp.concatenate(
        [head_h, h_o[_T], h_o[:_T].reshape(_T * _C, _D), tail_h], axis=0)
    merged_pos = jnp.concatenate(
        [head_p, p_o[_T], p_o[:_T].transpose(1, 0, 2).reshape(3, _T * _C),
         tail_p], axis=1)
    merged_cos = jnp.concatenate(
        [head_c, c_o[_T],
         c_o[:_T].transpose(1, 0, 2, 3).reshape(3, _T * _C, _HD), tail_c],
        axis=1)
    merged_sin = jnp.concatenate(
        [head_s, s_o[_T],
         s_o[:_T].transpose(1, 0, 2, 3).reshape(3, _T * _C, _HD), tail_s],
        axis=1)
    return merged_hidden, merged_pos, merged_cos, merged_sin


# speedup vs baseline: 7.2190x; 1.0626x over previous
import jax
import jax.numpy as jnp
from jax.experimental import pallas as pl
from jax.experimental.pallas import tpu as pltpu

_T, _N, _D, _HD = 32, 256, 2048, 128
_VS, _VE, _S = 64, 8256, 8320
_K = 5
_C = 64
_NSEG = 8
_TAU = 0.8
_EPS = 1e-6

_HI = jax.lax.Precision.HIGHEST


def _gram(f):
    return jax.lax.dot_general(
        f, f, (((1,), (1,)), ((), ())),
        precision=jax.lax.Precision.DEFAULT,
        preferred_element_type=jnp.float32)


def _dpc_labels(f, k, c):
    p = f.shape[0]
    sq = jnp.sum(f * f, axis=1)
    g = _gram(f)
    d2 = jnp.maximum(sq[:, None] + sq[None, :] - 2.0 * g, 0.0)
    rows = jax.lax.broadcasted_iota(jnp.int32, (p, p), 0)
    cols = jax.lax.broadcasted_iota(jnp.int32, (p, p), 1)
    work = d2
    acc = jnp.zeros((1, p), jnp.float32)
    for _ in range(k):
        m = jnp.min(work, axis=0, keepdims=True)
        acc = acc + m
        first = jnp.min(jnp.where(work == m, rows, p), axis=0, keepdims=True)
        work = jnp.where(rows == first, jnp.inf, work)
    density = jnp.exp(-(acc / float(k)))
    big = jnp.max(d2) + 1.0
    density_col = density.reshape(p, 1)
    higher = density_col > density
    delta = jnp.min(jnp.where(higher, d2, big), axis=0)
    dmax = jnp.max(d2, axis=0)
    delta = jnp.where(density[0] >= jnp.max(density), dmax, delta)
    score = (density[0] * delta)[None, :]
    score_col = score.reshape(p, 1)
    beats = (score_col > score) | ((score_col == score) & (rows < cols))
    slot = jnp.sum(beats.astype(jnp.float32), axis=0,
                   keepdims=True).astype(jnp.int32)
    slot_col = slot.reshape(p, 1)
    is_c_col = slot_col < c
    cand = jnp.where(is_c_col, d2, jnp.inf)
    minv = jnp.min(cand, axis=0, keepdims=True)
    lbl = jnp.min(jnp.where(is_c_col & (cand == minv),
                            jnp.broadcast_to(slot_col, (p, p)), p), axis=0)
    return lbl


def _stats_kernel(xa_ref, ms_ref, ff_ref, sim_ref, prev_ref):
    t = pl.program_id(0)
    xa = xa_ref[0]

    @pl.when(t == 0)
    def _():
        ms_ref[...] = xa

    @pl.when(t > 0)
    def _():
        ms_ref[...] = ms_ref[...] + xa

    ff_ref[0, 0] = jnp.sum(xa, axis=0) / float(_N)
    na = jnp.sqrt(jnp.sum(xa * xa, axis=1))
    xan = xa / (na[:, None] + _EPS)
    sim_ref[0, 0] = jnp.sum(prev_ref[...] * xan, axis=1)
    prev_ref[...] = xan


def _wstatic_kernel(ff_ref, sim_ref, w_ref):
    lbl = _dpc_labels(ff_ref[:, 0, :], _K, _NSEG)
    lbl_col = lbl[None, :].reshape(_T, 1)
    same_col = lbl_col[0:_T - 1] == lbl_col[1:_T]
    sim = sim_ref[0:_T - 1, 0, :]
    sim_eff = jnp.where(same_col, sim, 1.0)
    static_score = jnp.min(sim_eff, axis=0)
    w_ref[0, 0] = (static_score > _TAU).astype(jnp.float32)


def _merge_kernel(x_ref, ms_ref, w_ref, pos_ref, cos_ref, sin_ref,
                  h_ref, p_ref, c_ref, s_ref):
    gidx = pl.program_id(0)
    is_static = gidx == _T
    f = jnp.where(is_static, ms_ref[...] / float(_T), x_ref[0])
    wv = w_ref[0, 0]
    w = jnp.where(is_static, wv, 1.0 - wv)
    lbl = _dpc_labels(f, _K, _C)
    seg = jax.lax.broadcasted_iota(jnp.int32, (_C, _N), 0)
    onehot = (lbl[None, :] == seg).astype(jnp.float32)
    den = jnp.sum(onehot * w[None, :], axis=1) + _EPS
    for c0 in range(0, _D, 512):
        fwc = f[:, c0:c0 + 512] * w[:, None]
        shc = jax.lax.dot_general(onehot, fwc, (((1,), (0,)), ((), ())),
                                  precision=_HI,
                                  preferred_element_type=jnp.float32)
        h_ref[0, :, c0:c0 + 512] = shc / den[:, None]
    posw = pos_ref[0] * w[None, :]
    sp = jax.lax.dot_general(posw, onehot, (((1,), (1,)), ((), ())),
                             precision=_HI, preferred_element_type=jnp.float32)
    p_ref[0] = sp / den[None, :]
    for i in range(3):
        cw = cos_ref[i, 0] * w[:, None]
        sc = jax.lax.dot_general(onehot, cw, (((1,), (0,)), ((), ())),
                                 precision=_HI,
                                 preferred_element_type=jnp.float32)
        c_ref[0, i] = sc / den[:, None]
        sw = sin_ref[i, 0] * w[:, None]
        ss = jax.lax.dot_general(onehot, sw, (((1,), (0,)), ((), ())),
                                 precision=_HI,
                                 preferred_element_type=jnp.float32)
        s_ref[0, i] = ss / den[:, None]


def kernel(hidden_states, position_ids, cos, sin,
           visual_token_start, visual_token_end):
    vs = visual_token_start
    ve = visual_token_end
    nv = _VE - _VS
    nt = _S - _VE
    f32 = jnp.float32
    x3 = jax.lax.dynamic_slice(
        hidden_states, (0, vs, 0), (1, nv, _D))[0].reshape(_T, _N, _D)
    posv = (jax.lax.dynamic_slice(position_ids, (0, 0, vs), (3, 1, nv))[:, 0]
            .astype(f32).reshape(3, _T, _N).transpose(1, 0, 2))
    cosv = jax.lax.dynamic_slice(
        cos, (0, 0, vs, 0), (3, 1, nv, _HD))[:, 0].reshape(3, _T, _N, _HD)
    sinv = jax.lax.dynamic_slice(
        sin, (0, 0, vs, 0), (3, 1, nv, _HD))[:, 0].reshape(3, _T, _N, _HD)

    ms_sum, ff, sims = pl.pallas_call(
        _stats_kernel,
        grid=(_T,),
        in_specs=[
            pl.BlockSpec((1, _N, _D), lambda t: (t, 0, 0)),
        ],
        out_specs=[
            pl.BlockSpec((_N, _D), lambda t: (0, 0)),
            pl.BlockSpec((1, 1, _D), lambda t: (t, 0, 0)),
            pl.BlockSpec((1, 1, _N), lambda t: (jnp.maximum(t - 1, 0), 0, 0)),
        ],
        out_shape=[
            jax.ShapeDtypeStruct((_N, _D), f32),
            jax.ShapeDtypeStruct((_T, 1, _D), f32),
            jax.ShapeDtypeStruct((_T, 1, _N), f32),
        ],
        scratch_shapes=[pltpu.VMEM((_N, _D), f32)],
    )(x3)

    w_static = pl.pallas_call(
        _wstatic_kernel,
        out_shape=jax.ShapeDtypeStruct((1, 1, _N), f32),
    )(ff, sims)

    h_o, p_o, c_o, s_o = pl.pallas_call(
        _merge_kernel,
        grid=(_T + 1,),
        in_specs=[
            pl.BlockSpec((1, _N, _D), lambda g: (jnp.minimum(g, _T - 1), 0, 0)),
            pl.BlockSpec((_N, _D), lambda g: (0, 0)),
            pl.BlockSpec((1, 1, _N), lambda g: (0, 0, 0)),
            pl.BlockSpec((1, 3, _N), lambda g: (g % _T, 0, 0)),
            pl.BlockSpec((3, 1, _N, _HD), lambda g: (0, g % _T, 0, 0)),
            pl.BlockSpec((3, 1, _N, _HD), lambda g: (0, g % _T, 0, 0)),
        ],
        out_specs=[
            pl.BlockSpec((1, _C, _D), lambda g: (g, 0, 0)),
            pl.BlockSpec((1, 3, _C), lambda g: (g, 0, 0)),
            pl.BlockSpec((1, 3, _C, _HD), lambda g: (g, 0, 0, 0)),
            pl.BlockSpec((1, 3, _C, _HD), lambda g: (g, 0, 0, 0)),
        ],
        out_shape=[
            jax.ShapeDtypeStruct((_T + 1, _C, _D), f32),
            jax.ShapeDtypeStruct((_T + 1, 3, _C), f32),
            jax.ShapeDtypeStruct((_T + 1, 3, _C, _HD), f32),
            jax.ShapeDtypeStruct((_T + 1, 3, _C, _HD), f32),
        ],
    )(x3, ms_sum, w_static, posv, cosv, sinv)

    head_h = hidden_states[0, :_VS, :]
    tail_h = jax.lax.dynamic_slice(hidden_states, (0, ve, 0), (1, nt, _D))[0]
    head_p = position_ids[:, 0, :_VS].astype(f32)
    tail_p = jax.lax.dynamic_slice(
        position_ids, (0, 0, ve), (3, 1, nt))[:, 0].astype(f32)
    head_c = cos[:, 0, :_VS, :]
    tail_c = jax.lax.dynamic_slice(cos, (0, 0, ve, 0), (3, 1, nt, _HD))[:, 0]
    head_s = sin[:, 0, :_VS, :]
    tail_s = jax.lax.dynamic_slice(sin, (0, 0, ve, 0), (3, 1, nt, _HD))[:, 0]

    merged_hidden = jnp.concatenate(
        [head_h, h_o[_T], h_o[:_T].reshape(_T * _C, _D), tail_h], axis=0)
    merged_pos = jnp.concatenate(
        [head_p, p_o[_T], p_o[:_T].transpose(1, 0, 2).reshape(3, _T * _C),
         tail_p], axis=1)
    merged_cos = jnp.concatenate(
        [head_c, c_o[_T],
         c_o[:_T].transpose(1, 0, 2, 3).reshape(3, _T * _C, _HD), tail_c],
        axis=1)
    merged_sin = jnp.concatenate(
        [head_s, s_o[_T],
         s_o[:_T].transpose(1, 0, 2, 3).reshape(3, _T * _C, _HD), tail_s],
        axis=1)
    return merged_hidden, merged_pos, merged_cos, merged_sin


# speedup vs baseline: 8.5048x; 1.1781x over previous
import jax
import jax.numpy as jnp
from jax.experimental import pallas as pl
from jax.experimental.pallas import tpu as pltpu

_T, _N, _D, _HD = 32, 256, 2048, 128
_VS, _VE, _S = 64, 8256, 8320
_K = 5
_C = 64
_NSEG = 8
_TAU = 0.8
_EPS = 1e-6

_HI = jax.lax.Precision.HIGHEST


def _gram(f):
    return jax.lax.dot_general(
        f, f, (((1,), (1,)), ((), ())),
        precision=jax.lax.Precision.DEFAULT,
        preferred_element_type=jnp.float32)


def _dpc_labels(f, k, c):
    p = f.shape[0]
    sq = jnp.sum(f * f, axis=1)
    g = _gram(f)
    d2 = jnp.maximum(sq[:, None] + sq[None, :] - 2.0 * g, 0.0)
    rows = jax.lax.broadcasted_iota(jnp.int32, (p, p), 0)
    cols = jax.lax.broadcasted_iota(jnp.int32, (p, p), 1)
    work = d2
    acc = jnp.zeros((1, p), jnp.float32)
    for _ in range(k):
        m = jnp.min(work, axis=0, keepdims=True)
        acc = acc + m
        first = jnp.min(jnp.where(work == m, rows, p), axis=0, keepdims=True)
        work = jnp.where(rows == first, jnp.inf, work)
    density = jnp.exp(-(acc / float(k)))
    big = jnp.max(d2) + 1.0
    density_col = density.reshape(p, 1)
    higher = density_col > density
    delta = jnp.min(jnp.where(higher, d2, big), axis=0)
    dmax = jnp.max(d2, axis=0)
    delta = jnp.where(density[0] >= jnp.max(density), dmax, delta)
    score = (density[0] * delta)[None, :]
    score_col = score.reshape(p, 1)
    beats = (score_col > score) | ((score_col == score) & (rows < cols))
    slot = jnp.sum(beats.astype(jnp.float32), axis=0,
                   keepdims=True).astype(jnp.int32)
    slot_col = slot.reshape(p, 1)
    is_c_col = slot_col < c
    cand = jnp.where(is_c_col, d2, jnp.inf)
    minv = jnp.min(cand, axis=0, keepdims=True)
    lbl = jnp.min(jnp.where(is_c_col & (cand == minv),
                            jnp.broadcast_to(slot_col, (p, p)), p), axis=0)
    return lbl


def _stats_kernel(x0_ref, x1_ref, x2_ref, x3_ref, ms_ref, ff_ref, sim_ref,
                  prev_ref):
    t = pl.program_id(0)
    xa = jnp.concatenate(
        [x0_ref[0], x1_ref[0], x2_ref[0], x3_ref[0]], axis=0)

    @pl.when(t == 0)
    def _():
        ms_ref[...] = xa

    @pl.when(t > 0)
    def _():
        ms_ref[...] = ms_ref[...] + xa

    ff_ref[0, 0] = jnp.sum(xa, axis=0) / float(_N)
    na = jnp.sqrt(jnp.sum(xa * xa, axis=1))
    xan = xa / (na[:, None] + _EPS)
    sim_ref[0, 0] = jnp.sum(prev_ref[...] * xan, axis=1)
    prev_ref[...] = xan


def _wstatic_kernel(ff_ref, sim_ref, w_ref):
    lbl = _dpc_labels(ff_ref[:, 0, :], _K, _NSEG)
    lbl_col = lbl[None, :].reshape(_T, 1)
    same_col = lbl_col[0:_T - 1] == lbl_col[1:_T]
    sim = sim_ref[0:_T - 1, 0, :]
    sim_eff = jnp.where(same_col, sim, 1.0)
    static_score = jnp.min(sim_eff, axis=0)
    w_ref[0, 0] = (static_score > _TAU).astype(jnp.float32)


def _merge_kernel(x0_ref, x1_ref, x2_ref, x3_ref, ms_ref, w_ref,
                  pos_ref, cos_ref, sin_ref,
                  h_ref, p_ref, c_ref, s_ref):
    gidx = pl.program_id(0)
    is_static = gidx == _T
    xcat = jnp.concatenate(
        [x0_ref[0], x1_ref[0], x2_ref[0], x3_ref[0]], axis=0)
    f = jnp.where(is_static, ms_ref[...] / float(_T), xcat)
    wv = w_ref[0, 0]
    w = jnp.where(is_static, wv, 1.0 - wv)
    lbl = _dpc_labels(f, _K, _C)
    seg = jax.lax.broadcasted_iota(jnp.int32, (_C, _N), 0)
    onehot = (lbl[None, :] == seg).astype(jnp.float32)
    den = jnp.sum(onehot * w[None, :], axis=1) + _EPS
    for c0 in range(0, _D, 512):
        fwc = f[:, c0:c0 + 512] * w[:, None]
        shc = jax.lax.dot_general(onehot, fwc, (((1,), (0,)), ((), ())),
                                  precision=_HI,
                                  preferred_element_type=jnp.float32)
        h_ref[0, :, c0:c0 + 512] = shc / den[:, None]
    posw = pos_ref[0] * w[None, :]
    sp = jax.lax.dot_general(posw, onehot, (((1,), (1,)), ((), ())),
                             precision=_HI, preferred_element_type=jnp.float32)
    p_ref[0] = sp / den[None, :]
    for i in range(3):
        cw = cos_ref[i, 0] * w[:, None]
        sc = jax.lax.dot_general(onehot, cw, (((1,), (0,)), ((), ())),
                                 precision=_HI,
                                 preferred_element_type=jnp.float32)
        c_ref[0, i] = sc / den[:, None]
        sw = sin_ref[i, 0] * w[:, None]
        ss = jax.lax.dot_general(onehot, sw, (((1,), (0,)), ((), ())),
                                 precision=_HI,
                                 preferred_element_type=jnp.float32)
        s_ref[0, i] = ss / den[:, None]


def kernel(hidden_states, position_ids, cos, sin,
           visual_token_start, visual_token_end):
    vs = visual_token_start
    ve = visual_token_end
    nv = _VE - _VS
    nt = _S - _VE
    f32 = jnp.float32
    posv = (jax.lax.dynamic_slice(position_ids, (0, 0, vs), (3, 1, nv))[:, 0]
            .astype(f32).reshape(3, _T, _N).transpose(1, 0, 2))
    cosv = jax.lax.dynamic_slice(
        cos, (0, 0, vs, 0), (3, 1, nv, _HD))[:, 0].reshape(3, _T, _N, _HD)
    sinv = jax.lax.dynamic_slice(
        sin, (0, 0, vs, 0), (3, 1, nv, _HD))[:, 0].reshape(3, _T, _N, _HD)

    ms_sum, ff, sims = pl.pallas_call(
        _stats_kernel,
        grid=(_T,),
        in_specs=[
            pl.BlockSpec((1, _N // 4, _D), lambda t, q=q: (0, 4 * t + q + 1, 0))
            for q in range(4)
        ],
        out_specs=[
            pl.BlockSpec((_N, _D), lambda t: (0, 0)),
            pl.BlockSpec((1, 1, _D), lambda t: (t, 0, 0)),
            pl.BlockSpec((1, 1, _N), lambda t: (jnp.maximum(t - 1, 0), 0, 0)),
        ],
        out_shape=[
            jax.ShapeDtypeStruct((_N, _D), f32),
            jax.ShapeDtypeStruct((_T, 1, _D), f32),
            jax.ShapeDtypeStruct((_T, 1, _N), f32),
        ],
        scratch_shapes=[pltpu.VMEM((_N, _D), f32)],
    )(hidden_states, hidden_states, hidden_states, hidden_states)

    w_static = pl.pallas_call(
        _wstatic_kernel,
        out_shape=jax.ShapeDtypeStruct((1, 1, _N), f32),
    )(ff, sims)

    h_o, p_o, c_o, s_o = pl.pallas_call(
        _merge_kernel,
        grid=(_T + 1,),
        in_specs=[
            pl.BlockSpec(
                (1, _N // 4, _D),
                lambda g, q=q: (0, 4 * jnp.minimum(g, _T - 1) + q + 1, 0))
            for q in range(4)
        ] + [
            pl.BlockSpec((_N, _D), lambda g: (0, 0)),
            pl.BlockSpec((1, 1, _N), lambda g: (0, 0, 0)),
            pl.BlockSpec((1, 3, _N), lambda g: (g % _T, 0, 0)),
            pl.BlockSpec((3, 1, _N, _HD), lambda g: (0, g % _T, 0, 0)),
            pl.BlockSpec((3, 1, _N, _HD), lambda g: (0, g % _T, 0, 0)),
        ],
        out_specs=[
            pl.BlockSpec((1, _C, _D), lambda g: (g, 0, 0)),
            pl.BlockSpec((1, 3, _C), lambda g: (g, 0, 0)),
            pl.BlockSpec((1, 3, _C, _HD), lambda g: (g, 0, 0, 0)),
            pl.BlockSpec((1, 3, _C, _HD), lambda g: (g, 0, 0, 0)),
        ],
        out_shape=[
            jax.ShapeDtypeStruct((_T + 1, _C, _D), f32),
            jax.ShapeDtypeStruct((_T + 1, 3, _C), f32),
            jax.ShapeDtypeStruct((_T + 1, 3, _C, _HD), f32),
            jax.ShapeDtypeStruct((_T + 1, 3, _C, _HD), f32),
        ],
    )(hidden_states, hidden_states, hidden_states, hidden_states,
      ms_sum, w_static, posv, cosv, sinv)

    head_h = hidden_states[0, :_VS, :]
    tail_h = jax.lax.dynamic_slice(hidden_states, (0, ve, 0), (1, nt, _D))[0]
    head_p = position_ids[:, 0, :_VS].astype(f32)
    tail_p = jax.lax.dynamic_slice(
        position_ids, (0, 0, ve), (3, 1, nt))[:, 0].astype(f32)
    head_c = cos[:, 0, :_VS, :]
    tail_c = jax.lax.dynamic_slice(cos, (0, 0, ve, 0), (3, 1, nt, _HD))[:, 0]
    head_s = sin[:, 0, :_VS, :]
    tail_s = jax.lax.dynamic_slice(sin, (0, 0, ve, 0), (3, 1, nt, _HD))[:, 0]

    merged_hidden = jnp.concatenate(
        [head_h, h_o[_T], h_o[:_T].reshape(_T * _C, _D), tail_h], axis=0)
    merged_pos = jnp.concatenate(
        [head_p, p_o[_T], p_o[:_T].transpose(1, 0, 2).reshape(3, _T * _C),
         tail_p], axis=1)
    merged_cos = jnp.concatenate(
        [head_c, c_o[_T],
         c_o[:_T].transpose(1, 0, 2, 3).reshape(3, _T * _C, _HD), tail_c],
        axis=1)
    merged_sin = jnp.concatenate(
        [head_s, s_o[_T],
         s_o[:_T].transpose(1, 0, 2, 3).reshape(3, _T * _C, _HD), tail_s],
        axis=1)
    return merged_hidden, merged_pos, merged_cos, merged_sin
